# baseline (speedup 1.0000x reference)
import functools

import jax
import jax.numpy as jnp
from jax import lax
from jax.experimental import pallas as pl
from jax.experimental.pallas import tpu as pltpu
from jax.experimental.pallas import tpu_sc as plsc

B = 16384
CTX = 10
D = 64
NUM_ROWS = 1000000
NC = 2
NS = 16
NW = NC * NS
EPW = B // NW
CH = 32
NCHUNK = EPW // CH
RPC = CH * CTX
GSZ = 64
NG = RPC // GSZ
EPG = 8
NGRP = CH // EPG

TBLK = 32768
HB = TBLK // 2
NPAIR = ((NUM_ROWS + TBLK - 1) // TBLK) * HB


def _tr_body(x_ref, o_ref):
  y = x_ref[...].T
  o_ref[:, 0:D] = y[0:HB]
  o_ref[:, D:2 * D] = y[HB:TBLK]


def _to_row_major_pairs(table_t):
  n = table_t.shape[1]
  grid = (n + TBLK - 1) // TBLK
  return pl.pallas_call(
      _tr_body,
      grid=(grid,),
      in_specs=[pl.BlockSpec((D, TBLK), lambda i: (0, i))],
      out_specs=pl.BlockSpec((HB, 2 * D), lambda i: (i, 0)),
      out_shape=jax.ShapeDtypeStruct((grid * HB, 2 * D), jnp.float32),
  )(table_t)


def _sc_logits(batch, pos_flat, neg_flat, in_emb, out_emb):
  mesh = plsc.VectorSubcoreMesh(core_axis_name="c", subcore_axis_name="s",
                                num_cores=NC, num_subcores=NS)

  @functools.partial(
      pl.kernel,
      out_type=(jax.ShapeDtypeStruct((B * CTX,), jnp.float32),
                jax.ShapeDtypeStruct((B * CTX,), jnp.float32)),
      mesh=mesh,
      compiler_params=pltpu.CompilerParams(
          needs_layout_passes=False, use_tc_tiling_on_sc=False),
      scratch_types=[
          pltpu.VMEM((CH + 16,), jnp.int32),
          pltpu.VMEM((CH,), jnp.int32),
          pltpu.VMEM((RPC + 16,), jnp.int32),
          pltpu.VMEM((RPC,), jnp.int32),
          pltpu.VMEM((RPC + 16,), jnp.int32),
          pltpu.VMEM((RPC,), jnp.int32),
          pltpu.VMEM((CH, 2 * D), jnp.float32),
          pltpu.VMEM((RPC, 2 * D), jnp.float32),
          pltpu.VMEM((RPC, 2 * D), jnp.float32),
          pltpu.VMEM((EPW * CTX,), jnp.float32),
          pltpu.VMEM((EPW * CTX,), jnp.float32),
          pltpu.SemaphoreType.DMA,
          pltpu.SemaphoreType.DMA,
      ],
  )
  def k(batch_hbm, pos_hbm, neg_hbm, in_hbm, out_hbm, opos_hbm, oneg_hbm,
        bidx_v, bpair_v, sidxp_v, spairp_v, sidxn_v, spairn_v,
        u_v, rowsp_v, rowsn_v, plog_v, nlog_v, semp, semn):
    wid = lax.axis_index("s") * NC + lax.axis_index("c")
    lidx = lax.iota(jnp.int32, 16)

    def make_pairs(idx_ref, pair_ref, n16):
      def body(g, _):
        off = pl.multiple_of(g * 16, 16)
        v = idx_ref[pl.ds(off, 16)]
        pair_ref[pl.ds(off, 16)] = jnp.bitwise_or(
            jnp.left_shift(jnp.right_shift(v, 15), 14),
            jnp.bitwise_and(v, HB - 1))
        return 0
      lax.fori_loop(0, n16, body, 0)

    def chunk_body(c, _):
      bbase = pl.multiple_of(wid * EPW + c * CH, CH)
      pltpu.sync_copy(batch_hbm.at[pl.ds(bbase, CH)], bidx_v.at[pl.ds(0, CH)])
      make_pairs(bidx_v, bpair_v, CH // 16)
      ucp = pltpu.async_copy(in_hbm.at[bpair_v], u_v, semp)

      sbase = pl.multiple_of(wid * (EPW * CTX) + c * RPC, RPC)
      pltpu.sync_copy(pos_hbm.at[pl.ds(sbase, RPC)], sidxp_v.at[pl.ds(0, RPC)])
      make_pairs(sidxp_v, spairp_v, RPC // 16)
      cpsp = [
          pltpu.async_copy(
              out_hbm.at[spairp_v.at[pl.ds(g * GSZ, GSZ)]],
              rowsp_v.at[pl.ds(g * GSZ, GSZ)], semp)
          for g in range(NG)
      ]
      pltpu.sync_copy(neg_hbm.at[pl.ds(sbase, RPC)], sidxn_v.at[pl.ds(0, RPC)])
      make_pairs(sidxn_v, spairn_v, RPC // 16)
      cpsn = [
          pltpu.async_copy(
              out_hbm.at[spairn_v.at[pl.ds(g * GSZ, GSZ)]],
              rowsn_v.at[pl.ds(g * GSZ, GSZ)], semn)
          for g in range(NG)
      ]
      ucp.wait()
      for cp in cpsp:
        cp.wait()

      for t in (0, 1):
        sidx_v = sidxp_v if t == 0 else sidxn_v
        rows_v = rowsp_v if t == 0 else rowsn_v
        log_v = plog_v if t == 0 else nlog_v
        sign = 1.0 if t == 0 else -1.0

        def group_body(rg, _, sidx_v=sidx_v, rows_v=rows_v, log_v=log_v,
                       sign=sign, c=c):
          base_e = rg * EPG
          svec = jnp.zeros((16,), jnp.float32)
          for e in range(EPG):
            i = base_e + e
            ub = ((bidx_v[pl.ds(i, 16)][0] >> 14) & 1) * D
            u = [u_v[i, pl.ds(ub + kk * 16, 16)] for kk in range(4)]
            for p in range(CTX):
              r = e * CTX + p
              rr = i * CTX + p
              rb = ((sidx_v[pl.ds(rr, 16)][0] >> 14) & 1) * D
              acc = rows_v[rr, pl.ds(rb, 16)] * u[0]
              for kk in range(1, 4):
                acc = acc + rows_v[rr, pl.ds(rb + kk * 16, 16)] * u[kk]
              s = sign * jnp.sum(acc)
              svec = jnp.where(lidx == (r % 16), s, svec)
              if r % 16 == 15:
                off = pl.multiple_of(
                    c * RPC + rg * (EPG * CTX) + (r - 15), 16)
                log_v[pl.ds(off, 16)] = svec
          return 0

        lax.fori_loop(0, NGRP, group_body, 0)
        if t == 0:
          for cp in cpsn:
            cp.wait()
      return 0

    lax.fori_loop(0, NCHUNK, chunk_body, 0)
    obase = pl.multiple_of(wid * (EPW * CTX), EPW * CTX)
    pltpu.sync_copy(plog_v, opos_hbm.at[pl.ds(obase, EPW * CTX)])
    pltpu.sync_copy(nlog_v, oneg_hbm.at[pl.ds(obase, EPW * CTX)])

  return k(batch, pos_flat, neg_flat, in_emb, out_emb)


def _tc_body(p_ref, n_ref, o_ref):
  def log_sigmoid(x):
    return jnp.minimum(x, 0.0) - jnp.log1p(jnp.exp(-jnp.abs(x)))
  acc = jnp.sum(log_sigmoid(p_ref[...]), axis=1)
  acc = acc + jnp.sum(log_sigmoid(n_ref[...]), axis=1)
  o_ref[...] = -acc


def kernel(batch, pos_samps, neg_samps, in_emb, out_emb):
  batch = batch.astype(jnp.int32)
  pos_flat = pos_samps.astype(jnp.int32).reshape(-1)
  neg_flat = neg_samps.astype(jnp.int32).reshape(-1)

  in_rm = _to_row_major_pairs(in_emb.T)
  out_rm = _to_row_major_pairs(out_emb.T)

  plog, nlog = _sc_logits(batch, pos_flat, neg_flat, in_rm, out_rm)

  return pl.pallas_call(
      _tc_body,
      out_shape=jax.ShapeDtypeStruct((B,), jnp.float32),
  )(plog.reshape(B, CTX), nlog.reshape(B, CTX))


# speedup vs baseline: 1.0215x; 1.0215x over previous
import functools

import jax
import jax.numpy as jnp
from jax import lax
from jax.experimental import pallas as pl
from jax.experimental.pallas import tpu as pltpu
from jax.experimental.pallas import tpu_sc as plsc

B = 16384
CTX = 10
D = 64
NUM_ROWS = 1000000
NC = 2
NS = 16
NW = NC * NS
EPW = B // NW
CH = 64
NCHUNK = EPW // CH
RPC = CH * CTX
GSZ = 128
NG = RPC // GSZ
EPG = 8
NGRP = CH // EPG

TBLK = 32768
HB = TBLK // 2
NPAIR = ((NUM_ROWS + TBLK - 1) // TBLK) * HB


def _tr_body(x_ref, o_ref):
  y = x_ref[...].T
  o_ref[:, 0:D] = y[0:HB]
  o_ref[:, D:2 * D] = y[HB:TBLK]


def _to_row_major_pairs(table_t):
  n = table_t.shape[1]
  grid = (n + TBLK - 1) // TBLK
  return pl.pallas_call(
      _tr_body,
      grid=(grid,),
      in_specs=[pl.BlockSpec((D, TBLK), lambda i: (0, i))],
      out_specs=pl.BlockSpec((HB, 2 * D), lambda i: (i, 0)),
      out_shape=jax.ShapeDtypeStruct((grid * HB, 2 * D), jnp.float32),
  )(table_t)


def _sc_logits(batch, pos_flat, neg_flat, in_emb, out_emb):
  mesh = plsc.VectorSubcoreMesh(core_axis_name="c", subcore_axis_name="s",
                                num_cores=NC, num_subcores=NS)

  @functools.partial(
      pl.kernel,
      out_type=(jax.ShapeDtypeStruct((B * CTX,), jnp.float32),
                jax.ShapeDtypeStruct((B * CTX,), jnp.float32)),
      mesh=mesh,
      compiler_params=pltpu.CompilerParams(
          needs_layout_passes=False, use_tc_tiling_on_sc=False),
      scratch_types=[
          pltpu.VMEM((CH + 16,), jnp.int32),
          pltpu.VMEM((CH,), jnp.int32),
          pltpu.VMEM((RPC + 16,), jnp.int32),
          pltpu.VMEM((RPC,), jnp.int32),
          pltpu.VMEM((CH, 2 * D), jnp.float32),
          pltpu.VMEM((RPC, 2 * D), jnp.float32),
          pltpu.VMEM((EPW * CTX,), jnp.float32),
          pltpu.VMEM((EPW * CTX,), jnp.float32),
          pltpu.SemaphoreType.DMA,
      ],
  )
  def k(batch_hbm, pos_hbm, neg_hbm, in_hbm, out_hbm, opos_hbm, oneg_hbm,
        bidx_v, bpair_v, sidx_v, spair_v, u_v, rows_v, plog_v, nlog_v, sem):
    wid = lax.axis_index("s") * NC + lax.axis_index("c")
    lidx = lax.iota(jnp.int32, 16)

    def make_pairs(idx_ref, pair_ref, n16):
      def body(g, _):
        off = pl.multiple_of(g * 16, 16)
        v = idx_ref[pl.ds(off, 16)]
        pair_ref[pl.ds(off, 16)] = jnp.bitwise_or(
            jnp.left_shift(jnp.right_shift(v, 15), 14),
            jnp.bitwise_and(v, HB - 1))
        return 0
      lax.fori_loop(0, n16, body, 0)

    def chunk_body(c, _):
      bbase = pl.multiple_of(wid * EPW + c * CH, CH)
      pltpu.sync_copy(batch_hbm.at[pl.ds(bbase, CH)], bidx_v.at[pl.ds(0, CH)])
      make_pairs(bidx_v, bpair_v, CH // 16)
      pltpu.async_copy(in_hbm.at[bpair_v], u_v, sem).wait()

      for t in (0, 1):
        samp_hbm = pos_hbm if t == 0 else neg_hbm
        log_v = plog_v if t == 0 else nlog_v
        sign = 1.0 if t == 0 else -1.0
        sbase = pl.multiple_of(wid * (EPW * CTX) + c * RPC, RPC)
        pltpu.sync_copy(samp_hbm.at[pl.ds(sbase, RPC)],
                        sidx_v.at[pl.ds(0, RPC)])
        make_pairs(sidx_v, spair_v, RPC // 16)
        cps = [
            pltpu.async_copy(
                out_hbm.at[spair_v.at[pl.ds(g * GSZ, GSZ)]],
                rows_v.at[pl.ds(g * GSZ, GSZ)], sem)
            for g in range(NG)
        ]
        for cp in cps:
          cp.wait()

        def group_body(rg, _, log_v=log_v, sign=sign, c=c):
          base_e = rg * EPG
          svec = jnp.zeros((16,), jnp.float32)
          for e in range(EPG):
            i = base_e + e
            ub = ((bidx_v[pl.ds(i, 16)][0] >> 14) & 1) * D
            u = [u_v[i, pl.ds(ub + kk * 16, 16)] for kk in range(4)]
            for p in range(CTX):
              r = e * CTX + p
              rr = i * CTX + p
              rb = ((sidx_v[pl.ds(rr, 16)][0] >> 14) & 1) * D
              acc = rows_v[rr, pl.ds(rb, 16)] * u[0]
              for kk in range(1, 4):
                acc = acc + rows_v[rr, pl.ds(rb + kk * 16, 16)] * u[kk]
              s = sign * jnp.sum(acc)
              svec = jnp.where(lidx == (r % 16), s, svec)
              if r % 16 == 15:
                off = pl.multiple_of(
                    c * RPC + rg * (EPG * CTX) + (r - 15), 16)
                log_v[pl.ds(off, 16)] = svec
          return 0

        lax.fori_loop(0, NGRP, group_body, 0)
      return 0

    lax.fori_loop(0, NCHUNK, chunk_body, 0)
    obase = pl.multiple_of(wid * (EPW * CTX), EPW * CTX)
    pltpu.sync_copy(plog_v, opos_hbm.at[pl.ds(obase, EPW * CTX)])
    pltpu.sync_copy(nlog_v, oneg_hbm.at[pl.ds(obase, EPW * CTX)])

  return k(batch, pos_flat, neg_flat, in_emb, out_emb)


def _tc_body(p_ref, n_ref, o_ref):
  def log_sigmoid(x):
    return jnp.minimum(x, 0.0) - jnp.log1p(jnp.exp(-jnp.abs(x)))
  acc = jnp.sum(log_sigmoid(p_ref[...]), axis=1)
  acc = acc + jnp.sum(log_sigmoid(n_ref[...]), axis=1)
  o_ref[...] = -acc


def kernel(batch, pos_samps, neg_samps, in_emb, out_emb):
  batch = batch.astype(jnp.int32)
  pos_flat = pos_samps.astype(jnp.int32).reshape(-1)
  neg_flat = neg_samps.astype(jnp.int32).reshape(-1)

  in_rm = _to_row_major_pairs(in_emb.T)
  out_rm = _to_row_major_pairs(out_emb.T)

  plog, nlog = _sc_logits(batch, pos_flat, neg_flat, in_rm, out_rm)

  return pl.pallas_call(
      _tc_body,
      out_shape=jax.ShapeDtypeStruct((B,), jnp.float32),
  )(plog.reshape(B, CTX), nlog.reshape(B, CTX))
